# baseline (speedup 1.0000x reference)
import functools

import jax
import jax.numpy as jnp
from jax import lax
from jax.experimental import pallas as pl
from jax.experimental.pallas import tpu as pltpu
from jax.experimental.pallas import tpu_sc as plsc

_NC = 2
_NS = 16
_NW = _NC * _NS
_LANES = 16
_CHUNK_ROWS = 16
_TOK_NBUF = 4
_OUT_NBUF = 2
_POS_NBUF = 2
_SUPER = 8
_BS = 512
_SC_SEQ = 1024


def _sc_body(chunk, nslices, ngroups, batch, tok_hbm, pos_hbm, out_hbm,
             pos_v, tok_v, out_v, sem_tok, sem_out, sem_pos):
    c = lax.axis_index("c")
    s = lax.axis_index("s")
    wid = s * _NC + c
    base = wid * (ngroups * chunk)
    gps = _SUPER // batch
    nsuper = (ngroups * batch) // _SUPER

    def tok_pair(g, b, slot):
        return (tok_hbm.at[b, pl.ds(base + g * chunk, chunk)],
                tok_v.at[slot], sem_tok.at[slot])

    def pos_pair(g, slot):
        return (pos_hbm.at[pl.ds(base + g * chunk, chunk)],
                pos_v.at[slot], sem_pos.at[slot])

    def store_pair(g, b, slot):
        return (out_v.at[slot],
                out_hbm.at[b, pl.ds(base + g * chunk, chunk)],
                sem_out.at[slot])

    for r in range(_TOK_NBUF):
        pltpu.async_copy(*tok_pair(r // batch, r % batch, r))
    for g0 in range(_POS_NBUF):
        pltpu.async_copy(*pos_pair(g0, g0))

    def super_step(t, carry):
        for ul in range(_SUPER):
            b = ul % batch
            gl = ul // batch
            g = t * gps + gl
            tslot = ul % _TOK_NBUF
            oslot = ul % _OUT_NBUF

            pltpu.make_async_copy(*tok_pair(g, b, tslot)).wait()
            if b == 0:
                pltpu.make_async_copy(*pos_pair(g, gl % _POS_NBUF)).wait()

            r = ul - _OUT_NBUF
            gq, rem = divmod(r, _SUPER)
            pg = (t + gq) * gps + rem // batch
            pb = rem % batch
            if r >= 0:
                pltpu.make_async_copy(*store_pair(pg, pb, oslot)).wait()
            else:
                @pl.when(t > 0)
                def _():
                    pltpu.make_async_copy(*store_pair(pg, pb, oslot)).wait()

            @pl.loop(0, nslices, step=8)
            def _(j):
                o0 = j * _LANES
                toks = [tok_v[tslot, pl.ds(o0 + k * _LANES, _LANES)]
                        for k in range(8)]
                poss = [pos_v[gl % _POS_NBUF, pl.ds(o0 + k * _LANES, _LANES)]
                        for k in range(8)]
                for k in range(8):
                    out_v[oslot, pl.ds(o0 + k * _LANES, _LANES)] = (
                        toks[k] + poss[k])

            pltpu.async_copy(*store_pair(g, b, oslot))

            fg = g + _TOK_NBUF // batch
            fb = (ul + _TOK_NBUF) % batch
            @pl.when(fg < ngroups)
            def _():
                pltpu.async_copy(*tok_pair(fg, fb, tslot))

            if b == batch - 1:
                @pl.when(g + _POS_NBUF < ngroups)
                def _():
                    pltpu.async_copy(*pos_pair(g + _POS_NBUF, gl % _POS_NBUF))

        return carry

    lax.fori_loop(0, nsuper, super_step, 0)

    total_units = ngroups * batch
    for r in range(total_units - _OUT_NBUF, total_units):
        pltpu.make_async_copy(
            *store_pair(r // batch, r % batch, r % _OUT_NBUF)).wait()


def _sc_part(tok_tail, pos_tail):
    batch, seq, emb = tok_tail.shape
    n = seq * emb
    chunk = _CHUNK_ROWS * emb
    ngroups = (n // _NW) // chunk
    nslices = chunk // _LANES
    tok2 = tok_tail.reshape(batch, n)
    pos2 = pos_tail.reshape(n)
    mesh = plsc.VectorSubcoreMesh(core_axis_name="c", subcore_axis_name="s")
    f = pl.kernel(
        functools.partial(_sc_body, chunk, nslices, ngroups, batch),
        mesh=mesh,
        out_type=jax.ShapeDtypeStruct((batch, n), jnp.float32),
        scratch_types=[
            pltpu.VMEM((_POS_NBUF, chunk), jnp.float32),
            pltpu.VMEM((_TOK_NBUF, chunk), jnp.float32),
            pltpu.VMEM((_OUT_NBUF, chunk), jnp.float32),
            pltpu.SemaphoreType.DMA((_TOK_NBUF,)),
            pltpu.SemaphoreType.DMA((_OUT_NBUF,)),
            pltpu.SemaphoreType.DMA((_POS_NBUF,)),
        ],
    )
    return f(tok2, pos2).reshape(batch, seq, emb)


def _tc_body(tok_ref, pos_ref, out_ref):
    out_ref[...] = tok_ref[...] + pos_ref[...][None, :, :]


def _tc_part(tok_head, pos_head):
    batch, seq, emb = tok_head.shape
    grid = (seq // _BS, batch)
    return pl.pallas_call(
        _tc_body,
        grid=grid,
        in_specs=[
            pl.BlockSpec((1, _BS, emb), lambda i, b: (b, i, 0)),
            pl.BlockSpec((_BS, emb), lambda i, b: (i, 0)),
        ],
        out_specs=pl.BlockSpec((1, _BS, emb), lambda i, b: (b, i, 0)),
        out_shape=jax.ShapeDtypeStruct((batch, seq, emb), jnp.float32),
    )(tok_head, pos_head)


def kernel(token_embeddings, pos_embedding):
    batch, seq, emb = token_embeddings.shape
    s_tc = seq - _SC_SEQ
    out_tc = _tc_part(token_embeddings[:, :s_tc], pos_embedding[:s_tc])
    out_sc = _sc_part(token_embeddings[:, s_tc:], pos_embedding[s_tc:seq])
    return out_tc, out_sc


# speedup vs baseline: 1.0802x; 1.0802x over previous
import functools

import jax
import jax.numpy as jnp
from jax import lax
from jax.experimental import pallas as pl
from jax.experimental.pallas import tpu as pltpu
from jax.experimental.pallas import tpu_sc as plsc

_NC = 2
_NS = 16
_NW = _NC * _NS
_LANES = 16
_CHUNK_ROWS = 16
_TOK_NBUF = 4
_OUT_NBUF = 2
_POS_NBUF = 2
_SUPER = 8
_BS = 512
_SC_SEQ = 1024


def _sc_body(off, chunk, nslices, ngroups, batch, tok_hbm, pos_hbm, out_hbm,
             pos_v, tok_v, out_v, sem_tok, sem_out, sem_pos):
    c = lax.axis_index("c")
    s = lax.axis_index("s")
    wid = s * _NC + c
    base_out = wid * (ngroups * chunk)
    base = off + base_out
    gps = _SUPER // batch
    nsuper = (ngroups * batch) // _SUPER

    def tok_pair(g, b, slot):
        return (tok_hbm.at[b, pl.ds(base + g * chunk, chunk)],
                tok_v.at[slot], sem_tok.at[slot])

    def pos_pair(g, slot):
        return (pos_hbm.at[pl.ds(base + g * chunk, chunk)],
                pos_v.at[slot], sem_pos.at[slot])

    def store_pair(g, b, slot):
        return (out_v.at[slot],
                out_hbm.at[b, pl.ds(base_out + g * chunk, chunk)],
                sem_out.at[slot])

    for r in range(_TOK_NBUF):
        pltpu.async_copy(*tok_pair(r // batch, r % batch, r))
    for g0 in range(_POS_NBUF):
        pltpu.async_copy(*pos_pair(g0, g0))

    def super_step(t, carry):
        for ul in range(_SUPER):
            b = ul % batch
            gl = ul // batch
            g = t * gps + gl
            tslot = ul % _TOK_NBUF
            oslot = ul % _OUT_NBUF

            pltpu.make_async_copy(*tok_pair(g, b, tslot)).wait()
            if b == 0:
                pltpu.make_async_copy(*pos_pair(g, gl % _POS_NBUF)).wait()

            r = ul - _OUT_NBUF
            gq, rem = divmod(r, _SUPER)
            pg = (t + gq) * gps + rem // batch
            pb = rem % batch
            if r >= 0:
                pltpu.make_async_copy(*store_pair(pg, pb, oslot)).wait()
            else:
                @pl.when(t > 0)
                def _():
                    pltpu.make_async_copy(*store_pair(pg, pb, oslot)).wait()

            @pl.loop(0, nslices, step=8)
            def _(j):
                o0 = j * _LANES
                toks = [tok_v[tslot, pl.ds(o0 + k * _LANES, _LANES)]
                        for k in range(8)]
                poss = [pos_v[gl % _POS_NBUF, pl.ds(o0 + k * _LANES, _LANES)]
                        for k in range(8)]
                for k in range(8):
                    out_v[oslot, pl.ds(o0 + k * _LANES, _LANES)] = (
                        toks[k] + poss[k])

            pltpu.async_copy(*store_pair(g, b, oslot))

            fg = g + _TOK_NBUF // batch
            fb = (ul + _TOK_NBUF) % batch
            @pl.when(fg < ngroups)
            def _():
                pltpu.async_copy(*tok_pair(fg, fb, tslot))

            if b == batch - 1:
                @pl.when(g + _POS_NBUF < ngroups)
                def _():
                    pltpu.async_copy(*pos_pair(g + _POS_NBUF, gl % _POS_NBUF))

        return carry

    lax.fori_loop(0, nsuper, super_step, 0)

    total_units = ngroups * batch
    for r in range(total_units - _OUT_NBUF, total_units):
        pltpu.make_async_copy(
            *store_pair(r // batch, r % batch, r % _OUT_NBUF)).wait()


def _sc_part(tok2, pos2, sc_seq, emb, batch):
    n_tail = sc_seq * emb
    off = tok2.shape[1] - n_tail
    chunk = _CHUNK_ROWS * emb
    ngroups = (n_tail // _NW) // chunk
    nslices = chunk // _LANES
    mesh = plsc.VectorSubcoreMesh(core_axis_name="c", subcore_axis_name="s")
    f = pl.kernel(
        functools.partial(_sc_body, off, chunk, nslices, ngroups, batch),
        mesh=mesh,
        out_type=jax.ShapeDtypeStruct((batch, n_tail), jnp.float32),
        scratch_types=[
            pltpu.VMEM((_POS_NBUF, chunk), jnp.float32),
            pltpu.VMEM((_TOK_NBUF, chunk), jnp.float32),
            pltpu.VMEM((_OUT_NBUF, chunk), jnp.float32),
            pltpu.SemaphoreType.DMA((_TOK_NBUF,)),
            pltpu.SemaphoreType.DMA((_OUT_NBUF,)),
            pltpu.SemaphoreType.DMA((_POS_NBUF,)),
        ],
    )
    return f(tok2, pos2).reshape(batch, sc_seq, emb)


def _tc_body(tok_ref, pos_ref, out_ref):
    out_ref[...] = tok_ref[...] + pos_ref[...][None, :, :]


def _tc_part(tok, pos, s_tc):
    batch, seq, emb = tok.shape
    grid = (s_tc // _BS, batch)
    return pl.pallas_call(
        _tc_body,
        grid=grid,
        in_specs=[
            pl.BlockSpec((1, _BS, emb), lambda i, b: (b, i, 0)),
            pl.BlockSpec((_BS, emb), lambda i, b: (i, 0)),
        ],
        out_specs=pl.BlockSpec((1, _BS, emb), lambda i, b: (b, i, 0)),
        out_shape=jax.ShapeDtypeStruct((batch, seq, emb), jnp.float32),
    )(tok, pos)


def kernel(token_embeddings, pos_embedding):
    batch, seq, emb = token_embeddings.shape
    s_tc = seq - _SC_SEQ
    pos = pos_embedding[:seq]
    tok2 = token_embeddings.reshape(batch, seq * emb)
    pos2 = pos.reshape(seq * emb)
    out_tc = _tc_part(token_embeddings, pos, s_tc)
    out_sc = _sc_part(tok2, pos2, _SC_SEQ, emb, batch)
    return lax.dynamic_update_slice(out_tc, out_sc, (0, s_tc, 0))


# speedup vs baseline: 1.9490x; 1.8043x over previous
import functools

import jax
import jax.numpy as jnp
from jax import lax
from jax.experimental import pallas as pl
from jax.experimental.pallas import tpu as pltpu
from jax.experimental.pallas import tpu_sc as plsc

_NC = 2
_NS = 16
_NW = _NC * _NS
_LANES = 16
_CHUNK_ROWS = 16
_TOK_NBUF = 4
_OUT_NBUF = 2
_POS_NBUF = 2
_SUPER = 8
_BS = 512
_SC_SEQ = 1024


def _sc_body(s_tc, nslices, ngroups, batch, emb, tok_hbm, pos_hbm, out_hbm,
             pos_v, tok_v, out_v, sem_tok, sem_out, sem_pos):
    c = lax.axis_index("c")
    s = lax.axis_index("s")
    wid = s * _NC + c
    cr = _CHUNK_ROWS
    row_out0 = wid * (ngroups * cr)
    gps = _SUPER // batch
    nsuper = (ngroups * batch) // _SUPER

    def tok_pair(g, b, slot):
        return (tok_hbm.at[b, pl.ds(s_tc + (row_out0 + g * cr), cr), :],
                tok_v.at[slot], sem_tok.at[slot])

    def pos_pair(g, slot):
        return (pos_hbm.at[pl.ds(s_tc + (row_out0 + g * cr), cr), :],
                pos_v.at[slot], sem_pos.at[slot])

    def store_pair(g, b, slot):
        return (out_v.at[slot],
                out_hbm.at[b, pl.ds(row_out0 + g * cr, cr), :],
                sem_out.at[slot])

    for r in range(_TOK_NBUF):
        pltpu.async_copy(*tok_pair(r // batch, r % batch, r))
    for g0 in range(_POS_NBUF):
        pltpu.async_copy(*pos_pair(g0, g0))

    def super_step(t, carry):
        for ul in range(_SUPER):
            b = ul % batch
            gl = ul // batch
            g = t * gps + gl
            tslot = ul % _TOK_NBUF
            oslot = ul % _OUT_NBUF
            pslot = gl % _POS_NBUF

            pltpu.make_async_copy(*tok_pair(g, b, tslot)).wait()
            if b == 0:
                pltpu.make_async_copy(*pos_pair(g, pslot)).wait()

            r = ul - _OUT_NBUF
            gq, rem = divmod(r, _SUPER)
            pg = (t + gq) * gps + rem // batch
            pb = rem % batch
            if r >= 0:
                pltpu.make_async_copy(*store_pair(pg, pb, oslot)).wait()
            else:
                @pl.when(t > 0)
                def _():
                    pltpu.make_async_copy(*store_pair(pg, pb, oslot)).wait()

            @pl.loop(0, cr)
            def _(rr):
                @pl.loop(0, nslices // 8, step=1)
                def _(p):
                    o0 = p * (8 * _LANES)
                    toks = [tok_v[tslot, rr, pl.ds(o0 + k * _LANES, _LANES)]
                            for k in range(8)]
                    poss = [pos_v[pslot, rr, pl.ds(o0 + k * _LANES, _LANES)]
                            for k in range(8)]
                    for k in range(8):
                        out_v[oslot, rr, pl.ds(o0 + k * _LANES, _LANES)] = (
                            toks[k] + poss[k])

            pltpu.async_copy(*store_pair(g, b, oslot))

            fg = g + _TOK_NBUF // batch
            fb = (ul + _TOK_NBUF) % batch
            @pl.when(fg < ngroups)
            def _():
                pltpu.async_copy(*tok_pair(fg, fb, tslot))

            if b == batch - 1:
                @pl.when(g + _POS_NBUF < ngroups)
                def _():
                    pltpu.async_copy(*pos_pair(g + _POS_NBUF, pslot))

        return carry

    lax.fori_loop(0, nsuper, super_step, 0)

    total_units = ngroups * batch
    for r in range(total_units - _OUT_NBUF, total_units):
        pltpu.make_async_copy(
            *store_pair(r // batch, r % batch, r % _OUT_NBUF)).wait()


def _sc_part(tok, pos, s_tc, sc_seq):
    batch, seq, emb = tok.shape
    cr = _CHUNK_ROWS
    ngroups = (sc_seq // _NW) // cr
    nslices = (cr * emb) // (cr * _LANES)
    nslices = emb // _LANES
    mesh = plsc.VectorSubcoreMesh(core_axis_name="c", subcore_axis_name="s")
    f = pl.kernel(
        functools.partial(_sc_body, s_tc, nslices, ngroups, batch, emb),
        mesh=mesh,
        out_type=jax.ShapeDtypeStruct((batch, sc_seq, emb), jnp.float32),
        scratch_types=[
            pltpu.VMEM((_POS_NBUF, cr, emb), jnp.float32),
            pltpu.VMEM((_TOK_NBUF, cr, emb), jnp.float32),
            pltpu.VMEM((_OUT_NBUF, cr, emb), jnp.float32),
            pltpu.SemaphoreType.DMA((_TOK_NBUF,)),
            pltpu.SemaphoreType.DMA((_OUT_NBUF,)),
            pltpu.SemaphoreType.DMA((_POS_NBUF,)),
        ],
        compiler_params=pltpu.CompilerParams(use_tc_tiling_on_sc=True),
    )
    return f(tok, pos)


def _tc_body(tok_ref, pos_ref, out_ref):
    out_ref[...] = tok_ref[...] + pos_ref[...][None, :, :]


def _tc_part(tok, pos, s_tc):
    batch, seq, emb = tok.shape
    grid = (s_tc // _BS, batch)
    return pl.pallas_call(
        _tc_body,
        grid=grid,
        in_specs=[
            pl.BlockSpec((1, _BS, emb), lambda i, b: (b, i, 0)),
            pl.BlockSpec((_BS, emb), lambda i, b: (i, 0)),
        ],
        out_specs=pl.BlockSpec((1, _BS, emb), lambda i, b: (b, i, 0)),
        out_shape=jax.ShapeDtypeStruct((batch, seq, emb), jnp.float32),
    )(tok, pos)


def kernel(token_embeddings, pos_embedding):
    batch, seq, emb = token_embeddings.shape
    s_tc = seq - _SC_SEQ
    pos = pos_embedding[:seq]
    out_tc = _tc_part(token_embeddings, pos, s_tc)
    out_sc = _sc_part(token_embeddings, pos, s_tc, _SC_SEQ)
    return lax.dynamic_update_slice(out_tc, out_sc, (0, s_tc, 0))


# speedup vs baseline: 1.9855x; 1.0187x over previous
import functools

import jax
import jax.numpy as jnp
from jax import lax
from jax.experimental import pallas as pl
from jax.experimental.pallas import tpu as pltpu
from jax.experimental.pallas import tpu_sc as plsc

_NC = 2
_NS = 16
_NW = _NC * _NS
_LANES = 16
_CHUNK_ROWS = 8
_TOK_NBUF = 4
_OUT_NBUF = 2
_POS_NBUF = 2
_SUPER = 8
_BS = 512
_SC_SEQ = 512


def _sc_body(s_tc, nslices, ngroups, batch, emb, tok_hbm, pos_hbm, out_hbm,
             pos_v, tok_v, out_v, sem_tok, sem_out, sem_pos):
    c = lax.axis_index("c")
    s = lax.axis_index("s")
    wid = s * _NC + c
    cr = _CHUNK_ROWS
    row_out0 = wid * (ngroups * cr)
    gps = _SUPER // batch
    nsuper = (ngroups * batch) // _SUPER

    def tok_pair(g, b, slot):
        return (tok_hbm.at[b, pl.ds(s_tc + (row_out0 + g * cr), cr), :],
                tok_v.at[slot], sem_tok.at[slot])

    def pos_pair(g, slot):
        return (pos_hbm.at[pl.ds(s_tc + (row_out0 + g * cr), cr), :],
                pos_v.at[slot], sem_pos.at[slot])

    def store_pair(g, b, slot):
        return (out_v.at[slot],
                out_hbm.at[b, pl.ds(row_out0 + g * cr, cr), :],
                sem_out.at[slot])

    for r in range(_TOK_NBUF):
        pltpu.async_copy(*tok_pair(r // batch, r % batch, r))
    for g0 in range(_POS_NBUF):
        pltpu.async_copy(*pos_pair(g0, g0))

    def super_step(t, carry):
        for ul in range(_SUPER):
            b = ul % batch
            gl = ul // batch
            g = t * gps + gl
            tslot = ul % _TOK_NBUF
            oslot = ul % _OUT_NBUF
            pslot = gl % _POS_NBUF

            pltpu.make_async_copy(*tok_pair(g, b, tslot)).wait()
            if b == 0:
                pltpu.make_async_copy(*pos_pair(g, pslot)).wait()

            r = ul - _OUT_NBUF
            gq, rem = divmod(r, _SUPER)
            pg = (t + gq) * gps + rem // batch
            pb = rem % batch
            if r >= 0:
                pltpu.make_async_copy(*store_pair(pg, pb, oslot)).wait()
            else:
                @pl.when(t > 0)
                def _():
                    pltpu.make_async_copy(*store_pair(pg, pb, oslot)).wait()

            @pl.loop(0, cr)
            def _(rr):
                @pl.loop(0, nslices // 8, step=1)
                def _(p):
                    o0 = p * (8 * _LANES)
                    toks = [tok_v[tslot, rr, pl.ds(o0 + k * _LANES, _LANES)]
                            for k in range(8)]
                    poss = [pos_v[pslot, rr, pl.ds(o0 + k * _LANES, _LANES)]
                            for k in range(8)]
                    for k in range(8):
                        out_v[oslot, rr, pl.ds(o0 + k * _LANES, _LANES)] = (
                            toks[k] + poss[k])

            pltpu.async_copy(*store_pair(g, b, oslot))

            fg = g + _TOK_NBUF // batch
            fb = (ul + _TOK_NBUF) % batch
            @pl.when(fg < ngroups)
            def _():
                pltpu.async_copy(*tok_pair(fg, fb, tslot))

            if b == batch - 1:
                @pl.when(g + _POS_NBUF < ngroups)
                def _():
                    pltpu.async_copy(*pos_pair(g + _POS_NBUF, pslot))

        return carry

    lax.fori_loop(0, nsuper, super_step, 0)

    total_units = ngroups * batch
    for r in range(total_units - _OUT_NBUF, total_units):
        pltpu.make_async_copy(
            *store_pair(r // batch, r % batch, r % _OUT_NBUF)).wait()


def _sc_part(tok, pos, s_tc, sc_seq):
    batch, seq, emb = tok.shape
    cr = _CHUNK_ROWS
    ngroups = (sc_seq // _NW) // cr
    nslices = (cr * emb) // (cr * _LANES)
    nslices = emb // _LANES
    mesh = plsc.VectorSubcoreMesh(core_axis_name="c", subcore_axis_name="s")
    f = pl.kernel(
        functools.partial(_sc_body, s_tc, nslices, ngroups, batch, emb),
        mesh=mesh,
        out_type=jax.ShapeDtypeStruct((batch, sc_seq, emb), jnp.float32),
        scratch_types=[
            pltpu.VMEM((_POS_NBUF, cr, emb), jnp.float32),
            pltpu.VMEM((_TOK_NBUF, cr, emb), jnp.float32),
            pltpu.VMEM((_OUT_NBUF, cr, emb), jnp.float32),
            pltpu.SemaphoreType.DMA((_TOK_NBUF,)),
            pltpu.SemaphoreType.DMA((_OUT_NBUF,)),
            pltpu.SemaphoreType.DMA((_POS_NBUF,)),
        ],
        compiler_params=pltpu.CompilerParams(use_tc_tiling_on_sc=True),
    )
    return f(tok, pos)


def _tc_body(tok_ref, pos_ref, out_ref):
    out_ref[...] = tok_ref[...] + pos_ref[...][None, :, :]


def _tc_part(tok, pos, s_tc):
    batch, seq, emb = tok.shape
    grid = (s_tc // _BS, batch)
    return pl.pallas_call(
        _tc_body,
        grid=grid,
        in_specs=[
            pl.BlockSpec((1, _BS, emb), lambda i, b: (b, i, 0)),
            pl.BlockSpec((_BS, emb), lambda i, b: (i, 0)),
        ],
        out_specs=pl.BlockSpec((1, _BS, emb), lambda i, b: (b, i, 0)),
        out_shape=jax.ShapeDtypeStruct((batch, seq, emb), jnp.float32),
    )(tok, pos)


def kernel(token_embeddings, pos_embedding):
    batch, seq, emb = token_embeddings.shape
    s_tc = seq - _SC_SEQ
    pos = pos_embedding[:seq]
    out_tc = _tc_part(token_embeddings, pos, s_tc)
    out_sc = _sc_part(token_embeddings, pos, s_tc, _SC_SEQ)
    return lax.dynamic_update_slice(out_tc, out_sc, (0, s_tc, 0))


# speedup vs baseline: 2.3038x; 1.1603x over previous
import functools

import jax
import jax.numpy as jnp
from jax import lax
from jax.experimental import pallas as pl
from jax.experimental.pallas import tpu as pltpu
from jax.experimental.pallas import tpu_sc as plsc

_NC = 2
_NS = 16
_NW = _NC * _NS
_LANES = 16
_CHUNK_ROWS = 8
_TOK_NBUF = 4
_OUT_NBUF = 2
_POS_NBUF = 2
_SUPER = 8
_BS = 1920
_SC_SEQ = 512


def _sc_body(s_tc, nslices, ngroups, batch, emb, tok_hbm, pos_hbm, out_hbm,
             pos_v, tok_v, out_v, sem_tok, sem_out, sem_pos):
    c = lax.axis_index("c")
    s = lax.axis_index("s")
    wid = s * _NC + c
    cr = _CHUNK_ROWS
    row_out0 = wid * (ngroups * cr)
    gps = _SUPER // batch
    nsuper = (ngroups * batch) // _SUPER

    def tok_pair(g, b, slot):
        return (tok_hbm.at[b, pl.ds(s_tc + (row_out0 + g * cr), cr), :],
                tok_v.at[slot], sem_tok.at[slot])

    def pos_pair(g, slot):
        return (pos_hbm.at[pl.ds(s_tc + (row_out0 + g * cr), cr), :],
                pos_v.at[slot], sem_pos.at[slot])

    def store_pair(g, b, slot):
        return (out_v.at[slot],
                out_hbm.at[b, pl.ds(row_out0 + g * cr, cr), :],
                sem_out.at[slot])

    for r in range(_TOK_NBUF):
        pltpu.async_copy(*tok_pair(r // batch, r % batch, r))
    for g0 in range(_POS_NBUF):
        pltpu.async_copy(*pos_pair(g0, g0))

    def super_step(t, carry):
        for ul in range(_SUPER):
            b = ul % batch
            gl = ul // batch
            g = t * gps + gl
            tslot = ul % _TOK_NBUF
            oslot = ul % _OUT_NBUF
            pslot = gl % _POS_NBUF

            pltpu.make_async_copy(*tok_pair(g, b, tslot)).wait()
            if b == 0:
                pltpu.make_async_copy(*pos_pair(g, pslot)).wait()

            r = ul - _OUT_NBUF
            gq, rem = divmod(r, _SUPER)
            pg = (t + gq) * gps + rem // batch
            pb = rem % batch
            if r >= 0:
                pltpu.make_async_copy(*store_pair(pg, pb, oslot)).wait()
            else:
                @pl.when(t > 0)
                def _():
                    pltpu.make_async_copy(*store_pair(pg, pb, oslot)).wait()

            @pl.loop(0, cr)
            def _(rr):
                @pl.loop(0, nslices // 8, step=1)
                def _(p):
                    o0 = p * (8 * _LANES)
                    toks = [tok_v[tslot, rr, pl.ds(o0 + k * _LANES, _LANES)]
                            for k in range(8)]
                    poss = [pos_v[pslot, rr, pl.ds(o0 + k * _LANES, _LANES)]
                            for k in range(8)]
                    for k in range(8):
                        out_v[oslot, rr, pl.ds(o0 + k * _LANES, _LANES)] = (
                            toks[k] + poss[k])

            pltpu.async_copy(*store_pair(g, b, oslot))

            fg = g + _TOK_NBUF // batch
            fb = (ul + _TOK_NBUF) % batch
            @pl.when(fg < ngroups)
            def _():
                pltpu.async_copy(*tok_pair(fg, fb, tslot))

            if b == batch - 1:
                @pl.when(g + _POS_NBUF < ngroups)
                def _():
                    pltpu.async_copy(*pos_pair(g + _POS_NBUF, pslot))

        return carry

    lax.fori_loop(0, nsuper, super_step, 0)

    total_units = ngroups * batch
    for r in range(total_units - _OUT_NBUF, total_units):
        pltpu.make_async_copy(
            *store_pair(r // batch, r % batch, r % _OUT_NBUF)).wait()


def _sc_part(tok, pos, s_tc, sc_seq):
    batch, seq, emb = tok.shape
    cr = _CHUNK_ROWS
    ngroups = (sc_seq // _NW) // cr
    nslices = (cr * emb) // (cr * _LANES)
    nslices = emb // _LANES
    mesh = plsc.VectorSubcoreMesh(core_axis_name="c", subcore_axis_name="s")
    f = pl.kernel(
        functools.partial(_sc_body, s_tc, nslices, ngroups, batch, emb),
        mesh=mesh,
        out_type=jax.ShapeDtypeStruct((batch, sc_seq, emb), jnp.float32),
        scratch_types=[
            pltpu.VMEM((_POS_NBUF, cr, emb), jnp.float32),
            pltpu.VMEM((_TOK_NBUF, cr, emb), jnp.float32),
            pltpu.VMEM((_OUT_NBUF, cr, emb), jnp.float32),
            pltpu.SemaphoreType.DMA((_TOK_NBUF,)),
            pltpu.SemaphoreType.DMA((_OUT_NBUF,)),
            pltpu.SemaphoreType.DMA((_POS_NBUF,)),
        ],
        compiler_params=pltpu.CompilerParams(use_tc_tiling_on_sc=True),
    )
    return f(tok, pos)


def _tc_body(tok_ref, pos_ref, out_ref):
    out_ref[...] = tok_ref[...] + pos_ref[...][None, :, :]


def _tc_part(tok, pos, s_tc):
    batch, seq, emb = tok.shape
    grid = (s_tc // _BS, batch)
    return pl.pallas_call(
        _tc_body,
        grid=grid,
        in_specs=[
            pl.BlockSpec((1, _BS, emb), lambda i, b: (b, i, 0)),
            pl.BlockSpec((_BS, emb), lambda i, b: (i, 0)),
        ],
        out_specs=pl.BlockSpec((1, _BS, emb), lambda i, b: (b, i, 0)),
        out_shape=jax.ShapeDtypeStruct((batch, seq, emb), jnp.float32),
    )(tok, pos)


def kernel(token_embeddings, pos_embedding):
    batch, seq, emb = token_embeddings.shape
    s_tc = seq - _SC_SEQ
    pos = pos_embedding[:seq]
    out_tc = _tc_part(token_embeddings, pos, s_tc)
    out_sc = _sc_part(token_embeddings, pos, s_tc, _SC_SEQ)
    return lax.dynamic_update_slice(out_tc, out_sc, (0, s_tc, 0))


# speedup vs baseline: 2.3378x; 1.0147x over previous
import functools

import jax
import jax.numpy as jnp
from jax import lax
from jax.experimental import pallas as pl
from jax.experimental.pallas import tpu as pltpu
from jax.experimental.pallas import tpu_sc as plsc

_NC = 2
_NS = 16
_NW = _NC * _NS
_LANES = 16
_CHUNK_ROWS = 8
_TOK_NBUF = 4
_OUT_NBUF = 2
_POS_NBUF = 2
_SUPER = 8
_BS = 2560
_SC_SEQ = 512


def _sc_body(s_tc, nslices, ngroups, batch, emb, tok_hbm, pos_hbm, out_hbm,
             pos_v, tok_v, out_v, sem_tok, sem_out, sem_pos):
    c = lax.axis_index("c")
    s = lax.axis_index("s")
    wid = s * _NC + c
    cr = _CHUNK_ROWS
    row_out0 = wid * (ngroups * cr)
    gps = _SUPER // batch
    nsuper = (ngroups * batch) // _SUPER

    def tok_pair(g, b, slot):
        return (tok_hbm.at[b, pl.ds(s_tc + (row_out0 + g * cr), cr), :],
                tok_v.at[slot], sem_tok.at[slot])

    def pos_pair(g, slot):
        return (pos_hbm.at[pl.ds(s_tc + (row_out0 + g * cr), cr), :],
                pos_v.at[slot], sem_pos.at[slot])

    def store_pair(g, b, slot):
        return (out_v.at[slot],
                out_hbm.at[b, pl.ds(row_out0 + g * cr, cr), :],
                sem_out.at[slot])

    for r in range(_TOK_NBUF):
        pltpu.async_copy(*tok_pair(r // batch, r % batch, r))
    for g0 in range(_POS_NBUF):
        pltpu.async_copy(*pos_pair(g0, g0))

    def super_step(t, carry):
        for ul in range(_SUPER):
            b = ul % batch
            gl = ul // batch
            g = t * gps + gl
            tslot = ul % _TOK_NBUF
            oslot = ul % _OUT_NBUF
            pslot = gl % _POS_NBUF

            pltpu.make_async_copy(*tok_pair(g, b, tslot)).wait()
            if b == 0:
                pltpu.make_async_copy(*pos_pair(g, pslot)).wait()

            r = ul - _OUT_NBUF
            gq, rem = divmod(r, _SUPER)
            pg = (t + gq) * gps + rem // batch
            pb = rem % batch
            if r >= 0:
                pltpu.make_async_copy(*store_pair(pg, pb, oslot)).wait()
            else:
                @pl.when(t > 0)
                def _():
                    pltpu.make_async_copy(*store_pair(pg, pb, oslot)).wait()

            @pl.loop(0, cr)
            def _(rr):
                @pl.loop(0, nslices // 8, step=1)
                def _(p):
                    o0 = p * (8 * _LANES)
                    toks = [tok_v[tslot, rr, pl.ds(o0 + k * _LANES, _LANES)]
                            for k in range(8)]
                    poss = [pos_v[pslot, rr, pl.ds(o0 + k * _LANES, _LANES)]
                            for k in range(8)]
                    for k in range(8):
                        out_v[oslot, rr, pl.ds(o0 + k * _LANES, _LANES)] = (
                            toks[k] + poss[k])

            pltpu.async_copy(*store_pair(g, b, oslot))

            fg = g + _TOK_NBUF // batch
            fb = (ul + _TOK_NBUF) % batch
            @pl.when(fg < ngroups)
            def _():
                pltpu.async_copy(*tok_pair(fg, fb, tslot))

            if b == batch - 1:
                @pl.when(g + _POS_NBUF < ngroups)
                def _():
                    pltpu.async_copy(*pos_pair(g + _POS_NBUF, pslot))

        return carry

    lax.fori_loop(0, nsuper, super_step, 0)

    total_units = ngroups * batch
    for r in range(total_units - _OUT_NBUF, total_units):
        pltpu.make_async_copy(
            *store_pair(r // batch, r % batch, r % _OUT_NBUF)).wait()


def _sc_part(tok, pos, s_tc, sc_seq):
    batch, seq, emb = tok.shape
    cr = _CHUNK_ROWS
    ngroups = (sc_seq // _NW) // cr
    nslices = (cr * emb) // (cr * _LANES)
    nslices = emb // _LANES
    mesh = plsc.VectorSubcoreMesh(core_axis_name="c", subcore_axis_name="s")
    f = pl.kernel(
        functools.partial(_sc_body, s_tc, nslices, ngroups, batch, emb),
        mesh=mesh,
        out_type=jax.ShapeDtypeStruct((batch, sc_seq, emb), jnp.float32),
        scratch_types=[
            pltpu.VMEM((_POS_NBUF, cr, emb), jnp.float32),
            pltpu.VMEM((_TOK_NBUF, cr, emb), jnp.float32),
            pltpu.VMEM((_OUT_NBUF, cr, emb), jnp.float32),
            pltpu.SemaphoreType.DMA((_TOK_NBUF,)),
            pltpu.SemaphoreType.DMA((_OUT_NBUF,)),
            pltpu.SemaphoreType.DMA((_POS_NBUF,)),
        ],
        compiler_params=pltpu.CompilerParams(use_tc_tiling_on_sc=True),
    )
    return f(tok, pos)


def _tc_body(tok_ref, pos_ref, out_ref):
    out_ref[...] = tok_ref[...] + pos_ref[...][None, :, :]


def _tc_part(tok, pos, s_tc):
    batch, seq, emb = tok.shape
    grid = (s_tc // _BS, batch)
    return pl.pallas_call(
        _tc_body,
        grid=grid,
        in_specs=[
            pl.BlockSpec((1, _BS, emb), lambda i, b: (b, i, 0)),
            pl.BlockSpec((_BS, emb), lambda i, b: (i, 0)),
        ],
        out_specs=pl.BlockSpec((1, _BS, emb), lambda i, b: (b, i, 0)),
        out_shape=jax.ShapeDtypeStruct((batch, seq, emb), jnp.float32),
    )(tok, pos)


def kernel(token_embeddings, pos_embedding):
    batch, seq, emb = token_embeddings.shape
    s_tc = seq - _SC_SEQ
    pos = pos_embedding[:seq]
    out_tc = _tc_part(token_embeddings, pos, s_tc)
    out_sc = _sc_part(token_embeddings, pos, s_tc, _SC_SEQ)
    return lax.dynamic_update_slice(out_tc, out_sc, (0, s_tc, 0))


# speedup vs baseline: 2.3789x; 1.0176x over previous
import functools

import jax
import jax.numpy as jnp
from jax import lax
from jax.experimental import pallas as pl
from jax.experimental.pallas import tpu as pltpu
from jax.experimental.pallas import tpu_sc as plsc

_NC = 2
_NS = 16
_NW = _NC * _NS
_LANES = 16
_CHUNK_ROWS = 4
_TOK_NBUF = 4
_OUT_NBUF = 2
_POS_NBUF = 2
_SUPER = 8
_BS = 1984
_SC_SEQ = 256


def _sc_body(s_tc, nslices, ngroups, batch, emb, tok_hbm, pos_hbm, out_hbm,
             pos_v, tok_v, out_v, sem_tok, sem_out, sem_pos):
    c = lax.axis_index("c")
    s = lax.axis_index("s")
    wid = s * _NC + c
    cr = _CHUNK_ROWS
    row_out0 = wid * (ngroups * cr)
    gps = _SUPER // batch
    nsuper = (ngroups * batch) // _SUPER

    def tok_pair(g, b, slot):
        return (tok_hbm.at[b, pl.ds(s_tc + (row_out0 + g * cr), cr), :],
                tok_v.at[slot], sem_tok.at[slot])

    def pos_pair(g, slot):
        return (pos_hbm.at[pl.ds(s_tc + (row_out0 + g * cr), cr), :],
                pos_v.at[slot], sem_pos.at[slot])

    def store_pair(g, b, slot):
        return (out_v.at[slot],
                out_hbm.at[b, pl.ds(row_out0 + g * cr, cr), :],
                sem_out.at[slot])

    for r in range(_TOK_NBUF):
        pltpu.async_copy(*tok_pair(r // batch, r % batch, r))
    for g0 in range(_POS_NBUF):
        pltpu.async_copy(*pos_pair(g0, g0))

    def super_step(t, carry):
        for ul in range(_SUPER):
            b = ul % batch
            gl = ul // batch
            g = t * gps + gl
            tslot = ul % _TOK_NBUF
            oslot = ul % _OUT_NBUF
            pslot = gl % _POS_NBUF

            pltpu.make_async_copy(*tok_pair(g, b, tslot)).wait()
            if b == 0:
                pltpu.make_async_copy(*pos_pair(g, pslot)).wait()

            r = ul - _OUT_NBUF
            gq, rem = divmod(r, _SUPER)
            pg = (t + gq) * gps + rem // batch
            pb = rem % batch
            if r >= 0:
                pltpu.make_async_copy(*store_pair(pg, pb, oslot)).wait()
            else:
                @pl.when(t > 0)
                def _():
                    pltpu.make_async_copy(*store_pair(pg, pb, oslot)).wait()

            @pl.loop(0, cr)
            def _(rr):
                @pl.loop(0, nslices // 8, step=1)
                def _(p):
                    o0 = p * (8 * _LANES)
                    toks = [tok_v[tslot, rr, pl.ds(o0 + k * _LANES, _LANES)]
                            for k in range(8)]
                    poss = [pos_v[pslot, rr, pl.ds(o0 + k * _LANES, _LANES)]
                            for k in range(8)]
                    for k in range(8):
                        out_v[oslot, rr, pl.ds(o0 + k * _LANES, _LANES)] = (
                            toks[k] + poss[k])

            pltpu.async_copy(*store_pair(g, b, oslot))

            fg = g + _TOK_NBUF // batch
            fb = (ul + _TOK_NBUF) % batch
            @pl.when(fg < ngroups)
            def _():
                pltpu.async_copy(*tok_pair(fg, fb, tslot))

            if b == batch - 1:
                @pl.when(g + _POS_NBUF < ngroups)
                def _():
                    pltpu.async_copy(*pos_pair(g + _POS_NBUF, pslot))

        return carry

    lax.fori_loop(0, nsuper, super_step, 0)

    total_units = ngroups * batch
    for r in range(total_units - _OUT_NBUF, total_units):
        pltpu.make_async_copy(
            *store_pair(r // batch, r % batch, r % _OUT_NBUF)).wait()


def _sc_part(tok, pos, s_tc, sc_seq):
    batch, seq, emb = tok.shape
    cr = _CHUNK_ROWS
    ngroups = (sc_seq // _NW) // cr
    nslices = (cr * emb) // (cr * _LANES)
    nslices = emb // _LANES
    mesh = plsc.VectorSubcoreMesh(core_axis_name="c", subcore_axis_name="s")
    f = pl.kernel(
        functools.partial(_sc_body, s_tc, nslices, ngroups, batch, emb),
        mesh=mesh,
        out_type=jax.ShapeDtypeStruct((batch, sc_seq, emb), jnp.float32),
        scratch_types=[
            pltpu.VMEM((_POS_NBUF, cr, emb), jnp.float32),
            pltpu.VMEM((_TOK_NBUF, cr, emb), jnp.float32),
            pltpu.VMEM((_OUT_NBUF, cr, emb), jnp.float32),
            pltpu.SemaphoreType.DMA((_TOK_NBUF,)),
            pltpu.SemaphoreType.DMA((_OUT_NBUF,)),
            pltpu.SemaphoreType.DMA((_POS_NBUF,)),
        ],
        compiler_params=pltpu.CompilerParams(use_tc_tiling_on_sc=True),
    )
    return f(tok, pos)


def _tc_body(tok_ref, pos_ref, out_ref):
    out_ref[...] = tok_ref[...] + pos_ref[...][None, :, :]


def _tc_part(tok, pos, s_tc):
    batch, seq, emb = tok.shape
    grid = (s_tc // _BS, batch)
    return pl.pallas_call(
        _tc_body,
        grid=grid,
        in_specs=[
            pl.BlockSpec((1, _BS, emb), lambda i, b: (b, i, 0)),
            pl.BlockSpec((_BS, emb), lambda i, b: (i, 0)),
        ],
        out_specs=pl.BlockSpec((1, _BS, emb), lambda i, b: (b, i, 0)),
        out_shape=jax.ShapeDtypeStruct((batch, seq, emb), jnp.float32),
    )(tok, pos)


def kernel(token_embeddings, pos_embedding):
    batch, seq, emb = token_embeddings.shape
    s_tc = seq - _SC_SEQ
    pos = pos_embedding[:seq]
    out_tc = _tc_part(token_embeddings, pos, s_tc)
    out_sc = _sc_part(token_embeddings, pos, s_tc, _SC_SEQ)
    return lax.dynamic_update_slice(out_tc, out_sc, (0, s_tc, 0))
